# baseline (speedup 1.0000x reference)
import jax
import jax.numpy as jnp
from jax.experimental import pallas as pl

N, D_IN, D_OUT, E, R, K = 2048, 1024, 1024, 8, 32, 2
ER = E * R
TILE_N = 256


def _fused_kernel(x_ref, wt_ref, vt_ref, ucat_ref, bias_ref,
                  gates_ref, idx_ref, out_ref):
    xt = x_ref[...]
    xv = jnp.dot(xt, vt_ref[...], preferred_element_type=jnp.float32)

    col_e = jax.lax.broadcasted_iota(jnp.int32, (1, ER), 1) // R
    idx = idx_ref[...]
    g = gates_ref[...]
    w_full = jnp.zeros((xt.shape[0], ER), dtype=jnp.float32)
    for k in range(K):
        w_full = w_full + jnp.where(idx[:, k:k + 1] == col_e,
                                    g[:, k:k + 1], 0.0)
    t = (xv * w_full).astype(jnp.bfloat16)

    out = jnp.dot(xt, wt_ref[...], preferred_element_type=jnp.float32)
    out = out + jnp.dot(t, ucat_ref[...], preferred_element_type=jnp.float32)
    out_ref[...] = out + bias_ref[...]


@jax.jit
def kernel(x, weight_main, U_all, S_all, V_all, bias, top_k_gates,
           top_k_indices):
    xb = x.astype(jnp.bfloat16)
    wt = weight_main.T.astype(jnp.bfloat16)
    sv = (V_all * S_all[:, :, None]).reshape(ER, D_IN)
    vt = sv.T.astype(jnp.bfloat16)
    ucat = U_all.transpose(0, 2, 1).reshape(ER, D_OUT).astype(jnp.bfloat16)
    bias2 = bias.reshape(1, D_OUT)

    grid = (N // TILE_N,)
    out = pl.pallas_call(
        _fused_kernel,
        grid=grid,
        in_specs=[
            pl.BlockSpec((TILE_N, D_IN), lambda i: (i, 0)),
            pl.BlockSpec((D_IN, D_OUT), lambda i: (0, 0)),
            pl.BlockSpec((D_IN, ER), lambda i: (0, 0)),
            pl.BlockSpec((ER, D_OUT), lambda i: (0, 0)),
            pl.BlockSpec((1, D_OUT), lambda i: (0, 0)),
            pl.BlockSpec((TILE_N, K), lambda i: (i, 0)),
            pl.BlockSpec((TILE_N, K), lambda i: (i, 0)),
        ],
        out_specs=pl.BlockSpec((TILE_N, D_OUT), lambda i: (i, 0)),
        out_shape=jax.ShapeDtypeStruct((N, D_OUT), jnp.float32),
    )(xb, wt, vt, ucat, bias2, top_k_gates, top_k_indices)
    return out


# speedup vs baseline: 1.1179x; 1.1179x over previous
import jax
import jax.numpy as jnp
from jax.experimental import pallas as pl

N, D_IN, D_OUT, E, R, K = 2048, 1024, 1024, 8, 32, 2
ER = E * R
TILE_N = 2048


def _fused_kernel(x_ref, wt_ref, vt_ref, ucat_ref, bias_ref,
                  gates_ref, idx_ref, out_ref):
    xt = x_ref[...]
    xv = jnp.dot(xt, vt_ref[...], preferred_element_type=jnp.float32)

    col_e = jax.lax.broadcasted_iota(jnp.int32, (1, ER), 1) // R
    idx = idx_ref[...]
    g = gates_ref[...]
    w_full = jnp.zeros((xt.shape[0], ER), dtype=jnp.float32)
    for k in range(K):
        w_full = w_full + jnp.where(idx[:, k:k + 1] == col_e,
                                    g[:, k:k + 1], 0.0)
    t = xv * w_full

    out = jnp.dot(xt, wt_ref[...], preferred_element_type=jnp.float32)
    out = out + jnp.dot(t, ucat_ref[...], preferred_element_type=jnp.float32)
    out_ref[...] = out + bias_ref[...]


@jax.jit
def kernel(x, weight_main, U_all, S_all, V_all, bias, top_k_gates,
           top_k_indices):
    xb = x
    wt = weight_main.T
    sv = (V_all * S_all[:, :, None]).reshape(ER, D_IN)
    vt = sv.T
    ucat = U_all.transpose(0, 2, 1).reshape(ER, D_OUT)
    bias2 = bias.reshape(1, D_OUT)

    grid = (N // TILE_N,)
    out = pl.pallas_call(
        _fused_kernel,
        grid=grid,
        in_specs=[
            pl.BlockSpec((TILE_N, D_IN), lambda i: (i, 0)),
            pl.BlockSpec((D_IN, D_OUT), lambda i: (0, 0)),
            pl.BlockSpec((D_IN, ER), lambda i: (0, 0)),
            pl.BlockSpec((ER, D_OUT), lambda i: (0, 0)),
            pl.BlockSpec((1, D_OUT), lambda i: (0, 0)),
            pl.BlockSpec((TILE_N, K), lambda i: (i, 0)),
            pl.BlockSpec((TILE_N, K), lambda i: (i, 0)),
        ],
        out_specs=pl.BlockSpec((TILE_N, D_OUT), lambda i: (i, 0)),
        out_shape=jax.ShapeDtypeStruct((N, D_OUT), jnp.float32),
    )(xb, wt, vt, ucat, bias2, top_k_gates, top_k_indices)
    return out


# speedup vs baseline: 1.2743x; 1.1400x over previous
import jax
import jax.numpy as jnp
from jax.experimental import pallas as pl

N, D_IN, D_OUT, E, R, K = 2048, 1024, 1024, 8, 32, 2
ER = E * R
TILE_N = 2048


def _fused_kernel(x_ref, w_ref, v_ref, ucat_ref, s_ref, bias_ref,
                  gates_ref, idx_ref, out_ref):
    xt = x_ref[...]
    xv = jax.lax.dot_general(xt, v_ref[...], (((1,), (1,)), ((), ())),
                             preferred_element_type=jnp.float32)

    col_e = jax.lax.broadcasted_iota(jnp.int32, (1, ER), 1) // R
    idx = idx_ref[...]
    g = gates_ref[...]
    w_full = jnp.zeros((xt.shape[0], ER), dtype=jnp.float32)
    for k in range(K):
        w_full = w_full + jnp.where(idx[:, k:k + 1] == col_e,
                                    g[:, k:k + 1], 0.0)
    t = xv * (w_full * s_ref[...])

    out = jax.lax.dot_general(xt, w_ref[...], (((1,), (1,)), ((), ())),
                              preferred_element_type=jnp.float32)
    out = out + jnp.dot(t, ucat_ref[...], preferred_element_type=jnp.float32)
    out_ref[...] = out + bias_ref[...]


@jax.jit
def kernel(x, weight_main, U_all, S_all, V_all, bias, top_k_gates,
           top_k_indices):
    v2 = V_all.reshape(ER, D_IN)
    ucat = U_all.transpose(0, 2, 1).reshape(ER, D_OUT)
    s_flat = S_all.reshape(1, ER)
    bias2 = bias.reshape(1, D_OUT)

    grid = (N // TILE_N,)
    out = pl.pallas_call(
        _fused_kernel,
        grid=grid,
        in_specs=[
            pl.BlockSpec((TILE_N, D_IN), lambda i: (i, 0)),
            pl.BlockSpec((D_OUT, D_IN), lambda i: (0, 0)),
            pl.BlockSpec((ER, D_IN), lambda i: (0, 0)),
            pl.BlockSpec((ER, D_OUT), lambda i: (0, 0)),
            pl.BlockSpec((1, ER), lambda i: (0, 0)),
            pl.BlockSpec((1, D_OUT), lambda i: (0, 0)),
            pl.BlockSpec((TILE_N, K), lambda i: (i, 0)),
            pl.BlockSpec((TILE_N, K), lambda i: (i, 0)),
        ],
        out_specs=pl.BlockSpec((TILE_N, D_OUT), lambda i: (i, 0)),
        out_shape=jax.ShapeDtypeStruct((N, D_OUT), jnp.float32),
    )(x, weight_main, v2, ucat, s_flat, bias2, top_k_gates, top_k_indices)
    return out


# speedup vs baseline: 1.3758x; 1.0796x over previous
import jax
import jax.numpy as jnp
from jax.experimental import pallas as pl

N, D_IN, D_OUT, E, R, K = 2048, 1024, 1024, 8, 32, 2
ER = E * R
TILE_N = 512


def _fused_kernel(x_ref, w_ref, v_ref, ucat_ref, s_ref, bias_ref,
                  gates_ref, idx_ref, out_ref):
    xt = x_ref[...]
    xv = jax.lax.dot_general(xt, v_ref[...], (((1,), (1,)), ((), ())),
                             preferred_element_type=jnp.float32)

    col_e = jax.lax.broadcasted_iota(jnp.int32, (1, ER), 1) // R
    idx = idx_ref[...]
    g = gates_ref[...]
    w_full = jnp.zeros((xt.shape[0], ER), dtype=jnp.float32)
    for k in range(K):
        w_full = w_full + jnp.where(idx[:, k:k + 1] == col_e,
                                    g[:, k:k + 1], 0.0)
    t = xv * (w_full * s_ref[...])

    out = jax.lax.dot_general(xt, w_ref[...], (((1,), (1,)), ((), ())),
                              preferred_element_type=jnp.float32)
    out = out + jnp.dot(t, ucat_ref[...], preferred_element_type=jnp.float32)
    out_ref[...] = out + bias_ref[...]


@jax.jit
def kernel(x, weight_main, U_all, S_all, V_all, bias, top_k_gates,
           top_k_indices):
    v2 = V_all.reshape(ER, D_IN)
    ucat = U_all.transpose(0, 2, 1).reshape(ER, D_OUT)
    s_flat = S_all.reshape(1, ER)
    bias2 = bias.reshape(1, D_OUT)

    grid = (N // TILE_N,)
    out = pl.pallas_call(
        _fused_kernel,
        grid=grid,
        in_specs=[
            pl.BlockSpec((TILE_N, D_IN), lambda i: (i, 0)),
            pl.BlockSpec((D_OUT, D_IN), lambda i: (0, 0)),
            pl.BlockSpec((ER, D_IN), lambda i: (0, 0)),
            pl.BlockSpec((ER, D_OUT), lambda i: (0, 0)),
            pl.BlockSpec((1, ER), lambda i: (0, 0)),
            pl.BlockSpec((1, D_OUT), lambda i: (0, 0)),
            pl.BlockSpec((TILE_N, K), lambda i: (i, 0)),
            pl.BlockSpec((TILE_N, K), lambda i: (i, 0)),
        ],
        out_specs=pl.BlockSpec((TILE_N, D_OUT), lambda i: (i, 0)),
        out_shape=jax.ShapeDtypeStruct((N, D_OUT), jnp.float32),
    )(x, weight_main, v2, ucat, s_flat, bias2, top_k_gates, top_k_indices)
    return out
